# Initial kernel scaffold; baseline (speedup 1.0000x reference)
import jax
import jax.numpy as jnp
from jax.experimental import pallas as pl


def kernel(node_attrs, vectors, senders, receivers, W_emb1, W_emb2, W_v, W0a, W0b, Wg0, W1a, W1b, Wg1, W_out):
    raise NotImplementedError("write your pallas kernel here")


# speedup vs baseline: 7.4650x; 7.4650x over previous
import math

import jax
import jax.numpy as jnp
from jax.experimental import pallas as pl
from jax.experimental.pallas import tpu as pltpu

N_NODES = 10000
N_EDGES = 160000
D_ATTR = 128
HIDDEN = 128
D_V = 16
N_RADIAL = 8
CUTOFF = 1.0
AVG_NEIGH = 16.0

BE = 2000
_INV_SQRT2 = 1.0 / math.sqrt(2.0)
_INV_N = 1.0 / AVG_NEIGH
_F32 = jnp.float32


def _silu(x):
    return x * jax.nn.sigmoid(x)


def _dot(a, b):
    return jnp.dot(a, b, preferred_element_type=_F32)


def _embed_kernel(vec_ref, s_ref, r_ref, w1_ref, w2_ref, wv_ref,
                  x_ref, v_ref, cut_ref):
    v = vec_ref[...]
    vx, vy, vz = v[:, 0:1], v[:, 1:2], v[:, 2:3]
    d = jnp.sqrt(vx * vx + vy * vy + vz * vz)
    xc = jnp.clip(d / CUTOFF, 1e-6, 1.0)
    ns = (jax.lax.broadcasted_iota(jnp.int32, (1, N_RADIAL), 1) + 1).astype(_F32)
    rb = jnp.sqrt(2.0 / CUTOFF) * jnp.sin(ns * jnp.pi * xc) / xc
    xe = jnp.clip(d / CUTOFF, 0.0, 1.0)
    x6 = xe * xe * xe * xe * xe * xe
    cut = 1.0 - 28.0 * x6 + 48.0 * x6 * xe - 21.0 * x6 * xe * xe
    inv = 1.0 / jnp.maximum(d, 1e-6)
    ux, uy, uz = vx * inv, vy * inv, vz * inv
    one = jnp.ones_like(ux)
    sh = [one, ux, uy, uz, ux * uy, uy * uz, 3.0 * uz * uz - 1.0, ux * uz,
          ux * ux - uy * uy, uy * (3.0 * ux * ux - uy * uy), ux * uy * uz,
          uy * (5.0 * uz * uz - 1.0), uz * (5.0 * uz * uz - 3.0),
          ux * (5.0 * uz * uz - 1.0), uz * (ux * ux - uy * uy),
          ux * (ux * ux - 3.0 * uy * uy)]
    pre = (_dot(rb, w1_ref[0:N_RADIAL, :])
           + _dot(s_ref[...], w1_ref[N_RADIAL:N_RADIAL + D_ATTR, :])
           + _dot(r_ref[...], w1_ref[N_RADIAL + D_ATTR:, :]))
    x2 = _dot(_silu(pre), w2_ref[...]) * cut
    xv = _dot(x2, wv_ref[...])
    x_ref[...] = x2
    cut_ref[...] = cut
    for p in range(2):
        for i in range(8):
            v_ref[p, :, i * 16:(i + 1) * 16] = sh[p * 8 + i] * xv


def _layer_kernel(x_ref, v_ref, venv_ref, cut_ref, wa_ref, wb_ref, wg_ref,
                  xo_ref, vo_ref):
    x = x_ref[...]
    cut = cut_ref[...]
    t = jnp.zeros((x.shape[0], D_V), _F32)
    for p in range(2):
        for i in range(8):
            sl = slice(i * 16, (i + 1) * 16)
            t = t + v_ref[p, :, sl] * venv_ref[p, :, sl]
    t = t * _INV_N
    pre = _dot(x, wa_ref[0:HIDDEN, :]) + _dot(t, wa_ref[HIDDEN:HIDDEN + D_V, :])
    xn = (_dot(_silu(pre), wb_ref[...]) * cut + x) * _INV_SQRT2
    xo_ref[...] = xn
    g = _dot(xn, wg_ref[...])
    for p in range(2):
        for i in range(8):
            sl = slice(i * 16, (i + 1) * 16)
            vo_ref[p, :, sl] = (v_ref[p, :, sl]
                                + venv_ref[p, :, sl] * (_INV_N * g)) * _INV_SQRT2


def _final_kernel(x_ref, v_ref, venv_ref, cut_ref, wa_ref, wb_ref, wo_ref,
                  out_ref):
    x = x_ref[...]
    cut = cut_ref[...]
    t = jnp.zeros((x.shape[0], D_V), _F32)
    for p in range(2):
        for i in range(8):
            sl = slice(i * 16, (i + 1) * 16)
            t = t + v_ref[p, :, sl] * venv_ref[p, :, sl]
    t = t * _INV_N
    pre = _dot(x, wa_ref[0:HIDDEN, :]) + _dot(t, wa_ref[HIDDEN:HIDDEN + D_V, :])
    xn = (_dot(_silu(pre), wb_ref[...]) * cut + x) * _INV_SQRT2
    out_ref[...] = _dot(xn, wo_ref[...]) * cut


def _eblk(bs):
    return pl.BlockSpec(bs, lambda e: (e,) + (0,) * (len(bs) - 1))


def _wblk(bs):
    return pl.BlockSpec(bs, lambda e: (0,) * len(bs))


def _vblk():
    return pl.BlockSpec((2, BE, 128), lambda e: (0, e, 0))


def _tc_embed(vectors, S, R, W_emb1, W_emb2, W_v):
    return pl.pallas_call(
        _embed_kernel,
        grid=(N_EDGES // BE,),
        in_specs=[
            _eblk((BE, 3)),
            _eblk((BE, D_ATTR)),
            _eblk((BE, D_ATTR)),
            _wblk((N_RADIAL + 2 * D_ATTR, HIDDEN)),
            _wblk((HIDDEN, HIDDEN)),
            _wblk((HIDDEN, D_V)),
        ],
        out_specs=[
            _eblk((BE, HIDDEN)),
            _vblk(),
            _eblk((BE, 1)),
        ],
        out_shape=[
            jax.ShapeDtypeStruct((N_EDGES, HIDDEN), _F32),
            jax.ShapeDtypeStruct((2, N_EDGES, 128), _F32),
            jax.ShapeDtypeStruct((N_EDGES, 1), _F32),
        ],
    )(vectors, S, R, W_emb1, W_emb2, W_v)


def _tc_layer(x, V, Venv, cut, Wa, Wb, Wg):
    return pl.pallas_call(
        _layer_kernel,
        grid=(N_EDGES // BE,),
        in_specs=[
            _eblk((BE, HIDDEN)),
            _vblk(),
            _vblk(),
            _eblk((BE, 1)),
            _wblk((HIDDEN + D_V, HIDDEN)),
            _wblk((HIDDEN, HIDDEN)),
            _wblk((HIDDEN, D_V)),
        ],
        out_specs=[
            _eblk((BE, HIDDEN)),
            _vblk(),
        ],
        out_shape=[
            jax.ShapeDtypeStruct((N_EDGES, HIDDEN), _F32),
            jax.ShapeDtypeStruct((2, N_EDGES, 128), _F32),
        ],
    )(x, V, Venv, cut, Wa, Wb, Wg)


def _tc_final(x, V, Venv, cut, Wa, Wb, Wo):
    return pl.pallas_call(
        _final_kernel,
        grid=(N_EDGES // BE,),
        in_specs=[
            _eblk((BE, HIDDEN)),
            _vblk(),
            _vblk(),
            _eblk((BE, 1)),
            _wblk((HIDDEN + D_V, HIDDEN)),
            _wblk((HIDDEN, HIDDEN)),
            _wblk((HIDDEN, 1)),
        ],
        out_specs=_eblk((BE, 1)),
        out_shape=jax.ShapeDtypeStruct((N_EDGES, 1), _F32),
    )(x, V, Venv, cut, Wa, Wb, Wo)


def _gather_sr(node_attrs, senders, receivers):
    return node_attrs[senders], node_attrs[receivers]


def _segment_sum(V3d, senders):
    e0 = jax.ops.segment_sum(V3d[0], senders, num_segments=N_NODES)
    e1 = jax.ops.segment_sum(V3d[1], senders, num_segments=N_NODES)
    return jnp.stack([e0, e1])


def _gather_env(env3d, senders):
    return env3d[:, senders, :]


def kernel(node_attrs, vectors, senders, receivers, W_emb1, W_emb2, W_v,
           W0a, W0b, Wg0, W1a, W1b, Wg1, W_out):
    S, R = _gather_sr(node_attrs, senders, receivers)
    x0, V0, cut = _tc_embed(vectors, S, R, W_emb1, W_emb2, W_v)
    env0 = _segment_sum(V0, senders)
    Venv0 = _gather_env(env0, senders)
    x1, V1 = _tc_layer(x0, V0, Venv0, cut, W0a, W0b, Wg0)
    env1 = _segment_sum(V1, senders)
    Venv1 = _gather_env(env1, senders)
    return _tc_final(x1, V1, Venv1, cut, W1a, W1b, W_out)


# speedup vs baseline: 14.5014x; 1.9426x over previous
import functools
import math

import jax
import jax.numpy as jnp
from jax import lax
from jax.experimental import pallas as pl
from jax.experimental.pallas import tpu as pltpu
from jax.experimental.pallas import tpu_sc as plsc

N_NODES = 10000
N_EDGES = 160000
D_ATTR = 128
HIDDEN = 128
D_V = 16
N_RADIAL = 8
CUTOFF = 1.0
AVG_NEIGH = 16.0

BE = 2000
_INV_SQRT2 = 1.0 / math.sqrt(2.0)
_INV_N = 1.0 / AVG_NEIGH
_F32 = jnp.float32


def _silu(x):
    return x * jax.nn.sigmoid(x)


def _dot(a, b):
    return jnp.dot(a, b, preferred_element_type=_F32)


def _embed_kernel(vec_ref, s_ref, r_ref, w1_ref, w2_ref, wv_ref,
                  x_ref, v_ref, cut_ref):
    v = vec_ref[...]
    vx, vy, vz = v[:, 0:1], v[:, 1:2], v[:, 2:3]
    d = jnp.sqrt(vx * vx + vy * vy + vz * vz)
    xc = jnp.clip(d / CUTOFF, 1e-6, 1.0)
    ns = (jax.lax.broadcasted_iota(jnp.int32, (1, N_RADIAL), 1) + 1).astype(_F32)
    rb = jnp.sqrt(2.0 / CUTOFF) * jnp.sin(ns * jnp.pi * xc) / xc
    xe = jnp.clip(d / CUTOFF, 0.0, 1.0)
    x6 = xe * xe * xe * xe * xe * xe
    cut = 1.0 - 28.0 * x6 + 48.0 * x6 * xe - 21.0 * x6 * xe * xe
    inv = 1.0 / jnp.maximum(d, 1e-6)
    ux, uy, uz = vx * inv, vy * inv, vz * inv
    one = jnp.ones_like(ux)
    sh = [one, ux, uy, uz, ux * uy, uy * uz, 3.0 * uz * uz - 1.0, ux * uz,
          ux * ux - uy * uy, uy * (3.0 * ux * ux - uy * uy), ux * uy * uz,
          uy * (5.0 * uz * uz - 1.0), uz * (5.0 * uz * uz - 3.0),
          ux * (5.0 * uz * uz - 1.0), uz * (ux * ux - uy * uy),
          ux * (ux * ux - 3.0 * uy * uy)]
    pre = (_dot(rb, w1_ref[0:N_RADIAL, :])
           + _dot(s_ref[...], w1_ref[N_RADIAL:N_RADIAL + D_ATTR, :])
           + _dot(r_ref[...], w1_ref[N_RADIAL + D_ATTR:, :]))
    x2 = _dot(_silu(pre), w2_ref[...]) * cut
    xv = _dot(x2, wv_ref[...])
    x_ref[...] = x2
    cut_ref[...] = cut
    for p in range(2):
        for i in range(8):
            v_ref[p, :, i * 16:(i + 1) * 16] = sh[p * 8 + i] * xv


def _layer_kernel(x_ref, v_ref, venv_ref, cut_ref, wa_ref, wb_ref, wg_ref,
                  xo_ref, vo_ref):
    x = x_ref[...]
    cut = cut_ref[...]
    t = jnp.zeros((x.shape[0], D_V), _F32)
    for p in range(2):
        for i in range(8):
            sl = slice(i * 16, (i + 1) * 16)
            t = t + v_ref[p, :, sl] * venv_ref[p, :, sl]
    t = t * _INV_N
    pre = _dot(x, wa_ref[0:HIDDEN, :]) + _dot(t, wa_ref[HIDDEN:HIDDEN + D_V, :])
    xn = (_dot(_silu(pre), wb_ref[...]) * cut + x) * _INV_SQRT2
    xo_ref[...] = xn
    g = _dot(xn, wg_ref[...])
    for p in range(2):
        for i in range(8):
            sl = slice(i * 16, (i + 1) * 16)
            vo_ref[p, :, sl] = (v_ref[p, :, sl]
                                + venv_ref[p, :, sl] * (_INV_N * g)) * _INV_SQRT2


def _final_kernel(x_ref, v_ref, venv_ref, cut_ref, wa_ref, wb_ref, wo_ref,
                  out_ref):
    x = x_ref[...]
    cut = cut_ref[...]
    t = jnp.zeros((x.shape[0], D_V), _F32)
    for p in range(2):
        for i in range(8):
            sl = slice(i * 16, (i + 1) * 16)
            t = t + v_ref[p, :, sl] * venv_ref[p, :, sl]
    t = t * _INV_N
    pre = _dot(x, wa_ref[0:HIDDEN, :]) + _dot(t, wa_ref[HIDDEN:HIDDEN + D_V, :])
    xn = (_dot(_silu(pre), wb_ref[...]) * cut + x) * _INV_SQRT2
    out_ref[...] = _dot(xn, wo_ref[...]) * cut


def _eblk(bs):
    return pl.BlockSpec(bs, lambda e: (e,) + (0,) * (len(bs) - 1))


def _wblk(bs):
    return pl.BlockSpec(bs, lambda e: (0,) * len(bs))


def _vblk():
    return pl.BlockSpec((2, BE, 128), lambda e: (0, e, 0))


def _tc_embed(vectors, S, R, W_emb1, W_emb2, W_v):
    return pl.pallas_call(
        _embed_kernel,
        grid=(N_EDGES // BE,),
        in_specs=[
            _eblk((BE, 3)),
            _eblk((BE, D_ATTR)),
            _eblk((BE, D_ATTR)),
            _wblk((N_RADIAL + 2 * D_ATTR, HIDDEN)),
            _wblk((HIDDEN, HIDDEN)),
            _wblk((HIDDEN, D_V)),
        ],
        out_specs=[
            _eblk((BE, HIDDEN)),
            _vblk(),
            _eblk((BE, 1)),
        ],
        out_shape=[
            jax.ShapeDtypeStruct((N_EDGES, HIDDEN), _F32),
            jax.ShapeDtypeStruct((2, N_EDGES, 128), _F32),
            jax.ShapeDtypeStruct((N_EDGES, 1), _F32),
        ],
    )(vectors, S, R, W_emb1, W_emb2, W_v)


def _tc_layer(x, V, Venv, cut, Wa, Wb, Wg):
    return pl.pallas_call(
        _layer_kernel,
        grid=(N_EDGES // BE,),
        in_specs=[
            _eblk((BE, HIDDEN)),
            _vblk(),
            _vblk(),
            _eblk((BE, 1)),
            _wblk((HIDDEN + D_V, HIDDEN)),
            _wblk((HIDDEN, HIDDEN)),
            _wblk((HIDDEN, D_V)),
        ],
        out_specs=[
            _eblk((BE, HIDDEN)),
            _vblk(),
        ],
        out_shape=[
            jax.ShapeDtypeStruct((N_EDGES, HIDDEN), _F32),
            jax.ShapeDtypeStruct((2, N_EDGES, 128), _F32),
        ],
    )(x, V, Venv, cut, Wa, Wb, Wg)


def _tc_final(x, V, Venv, cut, Wa, Wb, Wo):
    return pl.pallas_call(
        _final_kernel,
        grid=(N_EDGES // BE,),
        in_specs=[
            _eblk((BE, HIDDEN)),
            _vblk(),
            _vblk(),
            _eblk((BE, 1)),
            _wblk((HIDDEN + D_V, HIDDEN)),
            _wblk((HIDDEN, HIDDEN)),
            _wblk((HIDDEN, 1)),
        ],
        out_specs=_eblk((BE, 1)),
        out_shape=jax.ShapeDtypeStruct((N_EDGES, 1), _F32),
    )(x, V, Venv, cut, Wa, Wb, Wo)


_ROWS = N_EDGES // 128
_SPAN = 80
_LAST = _ROWS - 15 * _SPAN
_ZSP = 624
_ZLAST = N_NODES - 15 * _ZSP

_SC_MESH = dict(core_axis_name="c", subcore_axis_name="s")


def _stage_idx(idx_src, start, idx_st):
    pltpu.sync_copy(idx_src.at[pl.ds(start, _SPAN)], idx_st)


def _sc_gather_sr(node_attrs, idx2):
    @functools.partial(
        pl.kernel,
        out_type=jax.ShapeDtypeStruct((2, N_EDGES, D_ATTR), _F32),
        mesh=plsc.VectorSubcoreMesh(**_SC_MESH),
        scratch_types=[
            pltpu.VMEM((_SPAN, 128), jnp.int32),
            pltpu.VMEM((128, D_ATTR), _F32),
        ],
    )
    def k(attrs_hbm, idx_hbm, out_hbm, idx_st, rows_v):
        c = lax.axis_index("c")
        s = lax.axis_index("s")
        start = s * _SPAN
        n = jnp.where(s < 15, _SPAN, _LAST)
        _stage_idx(idx_hbm.at[c], start, idx_st)

        @pl.loop(0, n)
        def _(j):
            pltpu.sync_copy(attrs_hbm.at[idx_st.at[j]], rows_v)
            pltpu.sync_copy(rows_v, out_hbm.at[c, pl.ds((start + j) * 128, 128)])

    out = k(node_attrs, idx2)
    return out[0], out[1]


def _sc_env_gather(V3d, senders2d, zblk):
    @functools.partial(
        pl.kernel,
        out_type=jax.ShapeDtypeStruct((2, N_EDGES, 128), _F32),
        mesh=plsc.VectorSubcoreMesh(**_SC_MESH),
        scratch_types=[
            pltpu.VMEM_SHARED((N_NODES, 128), _F32),
            pltpu.VMEM((_SPAN, 128), jnp.int32),
            pltpu.VMEM((128, 128), _F32),
        ],
    )
    def k(v_hbm, idx_hbm, z_hbm, venv_hbm, acc, idx_st, buf_v):
        c = lax.axis_index("c")
        s = lax.axis_index("s")
        start = s * _SPAN
        n = jnp.where(s < 15, _SPAN, _LAST)
        _stage_idx(idx_hbm, start, idx_st)

        @pl.when(s < 15)
        def _():
            pltpu.sync_copy(z_hbm.at[pl.ds(0, _ZSP)],
                            acc.at[pl.ds(s * _ZSP, _ZSP)])

        @pl.when(s >= 15)
        def _():
            pltpu.sync_copy(z_hbm, acc.at[pl.ds(15 * _ZSP, _ZLAST)])

        plsc.subcore_barrier()

        @pl.loop(0, n)
        def _(j):
            pltpu.sync_copy(v_hbm.at[c, pl.ds((start + j) * 128, 128)], buf_v)
            pltpu.sync_copy(buf_v, acc.at[idx_st.at[j]], add=True)

        plsc.subcore_barrier()

        @pl.loop(0, n)
        def _(j):
            pltpu.sync_copy(acc.at[idx_st.at[j]], buf_v)
            pltpu.sync_copy(buf_v, venv_hbm.at[c, pl.ds((start + j) * 128, 128)])

    return k(V3d, senders2d, zblk)


def kernel(node_attrs, vectors, senders, receivers, W_emb1, W_emb2, W_v,
           W0a, W0b, Wg0, W1a, W1b, Wg1, W_out):
    pad = ((0, 16 * _SPAN - _ROWS), (0, 0))
    senders2d = jnp.pad(senders.reshape(_ROWS, 128), pad)
    idx2 = jnp.stack([senders2d, jnp.pad(receivers.reshape(_ROWS, 128), pad)])
    zblk = jnp.zeros((_ZLAST, 128), _F32)
    S, R = _sc_gather_sr(node_attrs, idx2)
    x0, V0, cut = _tc_embed(vectors, S, R, W_emb1, W_emb2, W_v)
    Venv0 = _sc_env_gather(V0, senders2d, zblk)
    x1, V1 = _tc_layer(x0, V0, Venv0, cut, W0a, W0b, Wg0)
    Venv1 = _sc_env_gather(V1, senders2d, zblk)
    return _tc_final(x1, V1, Venv1, cut, W1a, W1b, W_out)
